# baseline (speedup 1.0000x reference)
import functools

import jax
import jax.numpy as jnp
from jax import lax
from jax.experimental import pallas as pl
from jax.experimental.pallas import tpu as pltpu
from jax.experimental.pallas import tpu_sc as plsc

_DIM = 32
_NC, _NS = 2, 16
_NW = _NC * _NS
_BCHUNK = 16


def _make_gather(batch, hist, vocab):
    assert batch % (_NW * _BCHUNK) == 0
    n_chunks = batch // (_NW * _BCHUNK)
    assert n_chunks % 2 == 0 and n_chunks >= 4
    b_per_w = n_chunks * _BCHUNK
    mesh = plsc.VectorSubcoreMesh(core_axis_name="c", subcore_axis_name="s")

    @functools.partial(
        pl.kernel,
        mesh=mesh,
        out_type=jax.ShapeDtypeStruct((hist, _DIM, batch), jnp.float32),
        compiler_params=pltpu.CompilerParams(use_tc_tiling_on_sc=False,
                                             needs_layout_passes=False),
        scratch_types=[
            pltpu.VMEM((b_per_w, hist), jnp.int32),
            pltpu.VMEM((_BCHUNK, hist, _DIM), jnp.float32),
            pltpu.VMEM((_BCHUNK, hist, _DIM), jnp.float32),
            pltpu.VMEM((hist, _DIM, _BCHUNK), jnp.float32),
            pltpu.SemaphoreType.DMA,
            pltpu.SemaphoreType.DMA,
        ],
    )
    def gather_kernel(idx_hbm, table_hbm, out_hbm, idx_v, rows0, rows1,
                      tbuf, sem0, sem1):
        wid = lax.axis_index("s") * _NC + lax.axis_index("c")
        b_base = wid * b_per_w
        pltpu.sync_copy(idx_hbm.at[pl.ds(b_base, b_per_w)], idx_v)

        bufs = (rows0, rows1)
        sems = (sem0, sem1)
        lane = lax.iota(jnp.int32, 16)

        def _copies(c, slot):
            for i in range(_BCHUNK):
                yield pltpu.make_async_copy(
                    table_hbm.at[idx_v.at[c * _BCHUNK + i]],
                    bufs[slot].at[i],
                    sems[slot],
                )

        def issue(c, slot):
            for cp in _copies(c, slot):
                cp.start()

        def drain(c, slot):
            for cp in _copies(c, slot):
                cp.wait()

        def flush(c, slot):
            def h_body(h, carry):
                hvec = jnp.full((16,), h, dtype=jnp.int32)
                for d in range(_DIM):
                    dvec = jnp.full((16,), d, dtype=jnp.int32)
                    tbuf[h, d, :] = plsc.load_gather(
                        bufs[slot], [lane, hvec, dvec])
                pltpu.sync_copy(
                    tbuf.at[h],
                    out_hbm.at[h, :, pl.ds(b_base + c * _BCHUNK, _BCHUNK)])
                return carry

            lax.fori_loop(0, hist, h_body, 0)

        issue(0, 0)
        issue(1, 1)
        drain(0, 0)
        flush(0, 0)

        def body(t, carry):
            c = 2 * t + 1
            issue(c + 1, 0)
            drain(c, 1)
            flush(c, 1)
            issue(c + 2, 1)
            drain(c + 1, 0)
            flush(c + 1, 0)
            return carry

        lax.fori_loop(0, n_chunks // 2 - 1, body, 0)
        drain(n_chunks - 1, 1)
        flush(n_chunks - 1, 1)

    return gather_kernel


def kernel(data, ivectors):
    b, h = data.shape
    vocab, dim = ivectors.shape
    assert dim == _DIM
    idx = data.astype(jnp.int32)
    out_t = _make_gather(b, h, vocab)(idx, ivectors)
    return out_t.transpose(2, 0, 1)


# speedup vs baseline: 1.3502x; 1.3502x over previous
import functools

import jax
import jax.numpy as jnp
from jax import lax
from jax.experimental import pallas as pl
from jax.experimental.pallas import tpu as pltpu
from jax.experimental.pallas import tpu_sc as plsc

_DIM = 32
_NC, _NS = 2, 16
_NW = _NC * _NS
_BCHUNK = 8


def _make_gather(batch, hist, vocab):
    assert batch % (_NW * _BCHUNK) == 0
    n_chunks = batch // (_NW * _BCHUNK)
    assert n_chunks % 2 == 0 and n_chunks >= 4
    b_per_w = n_chunks * _BCHUNK
    mesh = plsc.VectorSubcoreMesh(core_axis_name="c", subcore_axis_name="s")

    @functools.partial(
        pl.kernel,
        mesh=mesh,
        out_type=jax.ShapeDtypeStruct((batch, hist, _DIM), jnp.float32),
        compiler_params=pltpu.CompilerParams(use_tc_tiling_on_sc=False),
        scratch_types=[
            pltpu.VMEM((b_per_w, hist), jnp.int32),
            pltpu.VMEM((_BCHUNK, hist, _DIM), jnp.float32),
            pltpu.VMEM((_BCHUNK, hist, _DIM), jnp.float32),
            pltpu.SemaphoreType.DMA,
            pltpu.SemaphoreType.DMA,
        ],
    )
    def gather_kernel(idx_hbm, table_hbm, out_hbm, idx_v, rows0, rows1,
                      sem0, sem1):
        wid = lax.axis_index("s") * _NC + lax.axis_index("c")
        b_base = wid * b_per_w
        pltpu.sync_copy(idx_hbm.at[pl.ds(b_base, b_per_w)], idx_v)

        bufs = (rows0, rows1)
        sems = (sem0, sem1)

        def _copies(c, slot):
            for i in range(_BCHUNK):
                yield pltpu.make_async_copy(
                    table_hbm.at[idx_v.at[c * _BCHUNK + i]],
                    bufs[slot].at[i],
                    sems[slot],
                )

        def issue(c, slot):
            for cp in _copies(c, slot):
                cp.start()

        def drain(c, slot):
            for cp in _copies(c, slot):
                cp.wait()

        def flush(c, slot):
            pltpu.sync_copy(
                bufs[slot],
                out_hbm.at[pl.ds(b_base + c * _BCHUNK, _BCHUNK)])

        issue(0, 0)
        issue(1, 1)
        drain(0, 0)
        flush(0, 0)

        def body(t, carry):
            c = 2 * t + 1
            issue(c + 1, 0)
            drain(c, 1)
            flush(c, 1)
            issue(c + 2, 1)
            drain(c + 1, 0)
            flush(c + 1, 0)
            return carry

        lax.fori_loop(0, n_chunks // 2 - 1, body, 0)
        drain(n_chunks - 1, 1)
        flush(n_chunks - 1, 1)

    return gather_kernel


def kernel(data, ivectors):
    b, h = data.shape
    vocab, dim = ivectors.shape
    assert dim == _DIM
    idx = data.astype(jnp.int32)
    return _make_gather(b, h, vocab)(idx, ivectors)
